# Initial kernel scaffold; baseline (speedup 1.0000x reference)
import jax
import jax.numpy as jnp
from jax.experimental import pallas as pl


def kernel(x, gate_W, gate_b, W1, b1, W2, b2, ln_g, ln_b):
    raise NotImplementedError("write your pallas kernel here")


# speedup vs baseline: 1.0665x; 1.0665x over previous
import jax
import jax.numpy as jnp
from jax.experimental import pallas as pl
from jax.experimental.pallas import tpu as pltpu

B, S, D = 2, 2048, 768
E, H, TOP_K, TAU = 8, 2048, 2, 1.0
T = B * S
TB = 1024
NTB = T // TB


def _moe_body(x_ref, gum_ref, gw_ref, gb_ref, w1_ref, b1_ref, w2_ref, b2_ref,
              lng_ref, lnb_ref, out_ref, acc_ref):
    e = pl.program_id(1)
    x = x_ref[...]

    logits = jnp.dot(x, gw_ref[...], preferred_element_type=jnp.float32)
    noisy = logits + gb_ref[...] + gum_ref[...]
    v1 = jnp.max(noisy, axis=-1)
    i1 = jnp.argmax(noisy, axis=-1)
    cols = jax.lax.broadcasted_iota(jnp.int32, noisy.shape, 1)
    masked = jnp.where(cols == i1[:, None], -jnp.inf, noisy)
    v2 = jnp.max(masked, axis=-1)
    i2 = jnp.argmax(masked, axis=-1)
    t = jnp.exp(v2 - v1)
    w1g = 1.0 / (1.0 + t)
    w2g = t / (1.0 + t)
    we = jnp.where(i1 == e, w1g, jnp.where(i2 == e, w2g, 0.0))

    xb = x.astype(jnp.bfloat16)
    h = jnp.dot(xb, w1_ref[0], preferred_element_type=jnp.float32)
    h = jnp.maximum(h + b1_ref[0], 0.0).astype(jnp.bfloat16)
    o = jnp.dot(h, w2_ref[0], preferred_element_type=jnp.float32)
    contrib = we[:, None] * (o + b2_ref[0])

    @pl.when(e == 0)
    def _():
        acc_ref[...] = contrib

    @pl.when(e > 0)
    def _():
        acc_ref[...] = acc_ref[...] + contrib

    @pl.when(e == E - 1)
    def _():
        y = x + acc_ref[...]
        mu = jnp.mean(y, axis=-1, keepdims=True)
        var = jnp.mean((y - mu) ** 2, axis=-1, keepdims=True)
        out_ref[...] = ((y - mu) * jax.lax.rsqrt(var + 1e-5) * lng_ref[...]
                        + lnb_ref[...])


def kernel(x, gate_W, gate_b, W1, b1, W2, b2, ln_g, ln_b):
    nkey = jax.random.key(42)
    gumbel = -jnp.log(jax.random.exponential(nkey, (B, S, E), dtype=jnp.float32))

    xt = x.reshape(T, D)
    gum = gumbel.reshape(T, E)
    w1b = W1.astype(jnp.bfloat16)
    w2b = W2.astype(jnp.bfloat16)

    out = pl.pallas_call(
        _moe_body,
        grid=(NTB, E),
        in_specs=[
            pl.BlockSpec((TB, D), lambda t, e: (t, 0)),
            pl.BlockSpec((TB, E), lambda t, e: (t, 0)),
            pl.BlockSpec((D, E), lambda t, e: (0, 0)),
            pl.BlockSpec((1, E), lambda t, e: (0, 0)),
            pl.BlockSpec((1, D, H), lambda t, e: (e, 0, 0)),
            pl.BlockSpec((1, 1, H), lambda t, e: (e, 0, 0)),
            pl.BlockSpec((1, H, D), lambda t, e: (e, 0, 0)),
            pl.BlockSpec((1, 1, D), lambda t, e: (e, 0, 0)),
            pl.BlockSpec((1, D), lambda t, e: (0, 0)),
            pl.BlockSpec((1, D), lambda t, e: (0, 0)),
        ],
        out_specs=pl.BlockSpec((TB, D), lambda t, e: (t, 0)),
        out_shape=jax.ShapeDtypeStruct((T, D), jnp.float32),
        scratch_shapes=[pltpu.VMEM((TB, D), jnp.float32)],
    )(xt, gum, gate_W, gate_b.reshape(1, E), w1b, b1.reshape(E, 1, H), w2b,
      b2.reshape(E, 1, D), ln_g.reshape(1, D), ln_b.reshape(1, D))
    return out.reshape(B, S, D)
